# Initial kernel scaffold; baseline (speedup 1.0000x reference)
import jax
import jax.numpy as jnp
from jax.experimental import pallas as pl


def kernel(features, sentence_mask, init_rel, loop_rel, w_in, w_out, w_loop, w_rel, bias, bn_gamma, bn_beta, fc_W, fc_b, index, label, edges_type, edges):
    raise NotImplementedError("write your pallas kernel here")


# speedup vs baseline: 11.0609x; 11.0609x over previous
import functools

import jax
import jax.numpy as jnp
from jax import lax
from jax.experimental import pallas as pl
from jax.experimental.pallas import tpu as pltpu
from jax.experimental.pallas import tpu_sc as plsc

N = 10000
E = 320000
D = 128
H = 128
NCLS = 16
R = 16
B = 1024

NPAD = 10240
EH = E // 2
NCORES = 2
NTILES = 16
PER_TILE = EH // NTILES
CH = 128
ROWS = (PER_TILE + CH - 1) // CH
PT_PAD = ROWS * CH
OUT_SLICE = NPAD // NTILES
DUMMY = N
RB = 1024
NRB = NPAD // RB
SELW = 2 * B // (NCORES * NTILES)

_mesh = plsc.VectorSubcoreMesh(
    core_axis_name="c", subcore_axis_name="s",
    num_cores=NCORES, num_subcores=NTILES)


@functools.partial(
    pl.kernel,
    out_type=jax.ShapeDtypeStruct((NCORES, NPAD, D), jnp.float32),
    mesh=_mesh,
    scratch_types=[
        pltpu.VMEM_SHARED((NPAD, D), jnp.float32),
        pltpu.VMEM((ROWS, CH), jnp.int32),
        pltpu.VMEM((CH, D), jnp.float32),
        pltpu.VMEM((CH, D), jnp.float32),
    ],
)
def _degrees(src_hbm, ones_hbm, zer_hbm, deg_hbm, deg_sp, src_t, ones_v, buf_v):
    c = lax.axis_index("c")
    s = lax.axis_index("s")
    pltpu.sync_copy(src_hbm.at[c, s], src_t)
    pltpu.sync_copy(ones_hbm, ones_v)
    pltpu.sync_copy(zer_hbm, buf_v)
    for q in range(OUT_SLICE // CH):
        pltpu.sync_copy(buf_v, deg_sp.at[pl.ds(s * OUT_SLICE + q * CH, CH)])
    plsc.subcore_barrier()

    def body(j, carry):
        pltpu.sync_copy(ones_v, deg_sp.at[src_t.at[j]], add=True)
        return carry

    lax.fori_loop(0, ROWS, body, 0)
    plsc.subcore_barrier()
    for q in range(OUT_SLICE // CH):
        off = s * OUT_SLICE + q * CH
        pltpu.sync_copy(deg_sp.at[pl.ds(off, CH)], buf_v)
        pltpu.sync_copy(buf_v, deg_hbm.at[c, pl.ds(off, CH)])


def _build_body(x_ref, deg_ref, rel_ref, y_ref, dinv_ref):
    d = deg_ref[0][:, 0:1]
    dinv = jnp.where(d > 0.0, lax.rsqrt(d), 0.0)
    dinv_ref[0] = dinv
    y_ref[0, 0] = x_ref[...] * dinv * rel_ref[0]


def _build(xpad, deg, init_rel):
    return pl.pallas_call(
        _build_body,
        grid=(NRB, NCORES, R),
        in_specs=[
            pl.BlockSpec((RB, D), lambda i, c, r: (i, 0)),
            pl.BlockSpec((1, RB, D), lambda i, c, r: (c, i, 0)),
            pl.BlockSpec((1, 1, D), lambda i, c, r: (r, 0, 0)),
        ],
        out_specs=[
            pl.BlockSpec((1, 1, RB, D), lambda i, c, r: (c, r, i, 0)),
            pl.BlockSpec((1, RB, 1), lambda i, c, r: (c, i, 0)),
        ],
        out_shape=[
            jax.ShapeDtypeStruct((NCORES, R, NPAD, D), jnp.float32),
            jax.ShapeDtypeStruct((NCORES, NPAD, 1), jnp.float32),
        ],
    )(xpad, deg, init_rel)


@functools.partial(
    pl.kernel,
    out_type=jax.ShapeDtypeStruct((NCORES, NPAD, D), jnp.float32),
    mesh=_mesh,
    scratch_types=[
        pltpu.VMEM_SHARED((NPAD, D), jnp.float32),
        pltpu.VMEM((ROWS, CH), jnp.int32),
        pltpu.VMEM((ROWS, CH), jnp.int32),
        pltpu.VMEM((ROWS, CH), jnp.int32),
        pltpu.VMEM((CH, D), jnp.float32),
        pltpu.SemaphoreType.DMA,
    ],
)
def _aggregate(y_hbm, src_hbm, typ_hbm, dst_hbm, zer_hbm, acc_hbm,
               acc_sp, src_t, typ_t, dst_t, rows0, sem0):
    c = lax.axis_index("c")
    s = lax.axis_index("s")
    pltpu.sync_copy(src_hbm.at[c, s], src_t)
    pltpu.sync_copy(typ_hbm.at[c, s], typ_t)
    pltpu.sync_copy(dst_hbm.at[c, s], dst_t)
    pltpu.sync_copy(zer_hbm, rows0)
    for q in range(OUT_SLICE // CH):
        pltpu.sync_copy(rows0, acc_sp.at[pl.ds(s * OUT_SLICE + q * CH, CH)])
    base = (c * R) * NPAD

    def fidx(j, carry):
        for k in range(CH // 16):
            sl = pl.ds(k * 16, 16)
            src_t[j, sl] = typ_t[j, sl] * NPAD + src_t[j, sl] + base
        return carry

    lax.fori_loop(0, ROWS, fidx, 0)
    plsc.subcore_barrier()

    def chunk(j, carry):
        pltpu.async_copy(y_hbm.at[src_t.at[j]], rows0, sem0).wait()
        pltpu.sync_copy(rows0, acc_sp.at[dst_t.at[j]], add=True)
        return carry

    lax.fori_loop(0, ROWS, chunk, 0)
    plsc.subcore_barrier()
    for q in range(OUT_SLICE // CH):
        off = s * OUT_SLICE + q * CH
        pltpu.sync_copy(acc_sp.at[pl.ds(off, CH)], rows0)
        pltpu.sync_copy(rows0, acc_hbm.at[c, pl.ds(off, CH)])


def _dense_body(ain_ref, aout_ref, x_ref, din_ref, dout_ref,
                win_ref, wout_ref, wloop_ref, lrel_ref, bias_ref,
                outpre_ref, stats_ref, sacc):
    i = pl.program_id(0)
    a0 = ain_ref[...] * din_ref[...]
    a1 = aout_ref[...] * dout_ref[...]
    xl = x_ref[...] * lrel_ref[...]
    pre = (jnp.dot(a0, win_ref[...], preferred_element_type=jnp.float32)
           + jnp.dot(a1, wout_ref[...], preferred_element_type=jnp.float32)
           + jnp.dot(xl, wloop_ref[...], preferred_element_type=jnp.float32))
    pre = pre * (1.0 / 3.0) + bias_ref[...]
    outpre_ref[...] = pre
    rowid = i * RB + lax.broadcasted_iota(jnp.int32, (RB, 1), 0)
    prem = jnp.where(rowid < N, pre, 0.0)

    @pl.when(i == 0)
    def _():
        sacc[...] = jnp.zeros((8, 128), jnp.float32)

    s0 = jnp.sum(prem, axis=0, keepdims=True)
    s1 = jnp.sum(prem * prem, axis=0, keepdims=True)
    sacc[0:1, :] = sacc[0:1, :] + s0
    sacc[1:2, :] = sacc[1:2, :] + s1

    @pl.when(i == NRB - 1)
    def _():
        stats_ref[...] = sacc[...]


def _dense(acc_in, acc_out, xpad, dinv_in, dinv_out,
           w_in, w_out, w_loop, loop_rel, bias2):
    return pl.pallas_call(
        _dense_body,
        grid=(NRB,),
        in_specs=[
            pl.BlockSpec((RB, D), lambda i: (i, 0)),
            pl.BlockSpec((RB, D), lambda i: (i, 0)),
            pl.BlockSpec((RB, D), lambda i: (i, 0)),
            pl.BlockSpec((RB, 1), lambda i: (i, 0)),
            pl.BlockSpec((RB, 1), lambda i: (i, 0)),
            pl.BlockSpec((D, H), lambda i: (0, 0)),
            pl.BlockSpec((D, H), lambda i: (0, 0)),
            pl.BlockSpec((D, H), lambda i: (0, 0)),
            pl.BlockSpec((1, D), lambda i: (0, 0)),
            pl.BlockSpec((1, H), lambda i: (0, 0)),
        ],
        out_specs=[
            pl.BlockSpec((RB, H), lambda i: (i, 0)),
            pl.BlockSpec((8, 128), lambda i: (0, 0)),
        ],
        out_shape=[
            jax.ShapeDtypeStruct((NPAD, H), jnp.float32),
            jax.ShapeDtypeStruct((8, 128), jnp.float32),
        ],
        scratch_shapes=[pltpu.VMEM((8, 128), jnp.float32)],
    )(acc_in, acc_out, xpad, dinv_in, dinv_out, w_in, w_out, w_loop,
      loop_rel, bias2)


@functools.partial(
    pl.kernel,
    out_type=jax.ShapeDtypeStruct((2 * B, D), jnp.float32),
    mesh=_mesh,
    scratch_types=[
        pltpu.VMEM((SELW,), jnp.int32),
        pltpu.VMEM((SELW, D), jnp.float32),
        pltpu.SemaphoreType.DMA,
    ],
)
def _select_rows(xpre_hbm, idx_hbm, sel_hbm, idx_v, rows_v, sem):
    c = lax.axis_index("c")
    s = lax.axis_index("s")
    w = s * NCORES + c
    pltpu.sync_copy(idx_hbm.at[w], idx_v)
    pltpu.async_copy(xpre_hbm.at[idx_v], rows_v, sem).wait()
    pltpu.sync_copy(rows_v, sel_hbm.at[pl.ds(w * SELW, SELW)])


def _head_body(sel_ref, stats_ref, gam_ref, bet_ref, w0_ref, w1_ref,
               fcb_ref, lab_ref, loss_ref, logits_ref):
    mean = stats_ref[0:1, :] * (1.0 / N)
    var = stats_ref[1:2, :] * (1.0 / N) - mean * mean
    rs = lax.rsqrt(var + 1e-5)
    xt = jnp.tanh((sel_ref[...] - mean) * rs * gam_ref[...] + bet_ref[...])
    h0 = xt[0:B]
    h1 = xt[B:2 * B]
    lg = (jnp.dot(h0, w0_ref[...], preferred_element_type=jnp.float32)
          + jnp.dot(h1, w1_ref[...], preferred_element_type=jnp.float32)
          + fcb_ref[...])
    m = jnp.max(lg, axis=1, keepdims=True)
    lse = m + jnp.log(jnp.sum(jnp.exp(lg - m), axis=1, keepdims=True))
    lp = lg - lse
    onehot = lax.broadcasted_iota(jnp.int32, (B, NCLS), 1) == lab_ref[...]
    loss_ref[...] = (-jnp.sum(jnp.where(onehot, lp, 0.0)) * (1.0 / B)
                     ).reshape(1, 1)
    logits_ref[...] = lg


def _head(sel, stats, gam2, bet2, w0t, w1t, fcb2, lab2):
    return pl.pallas_call(
        _head_body,
        in_specs=[
            pl.BlockSpec((2 * B, D), lambda: (0, 0)),
            pl.BlockSpec((8, 128), lambda: (0, 0)),
            pl.BlockSpec((1, H), lambda: (0, 0)),
            pl.BlockSpec((1, H), lambda: (0, 0)),
            pl.BlockSpec((H, NCLS), lambda: (0, 0)),
            pl.BlockSpec((H, NCLS), lambda: (0, 0)),
            pl.BlockSpec((1, NCLS), lambda: (0, 0)),
            pl.BlockSpec((B, 1), lambda: (0, 0)),
        ],
        out_specs=[
            pl.BlockSpec((1, 1), lambda: (0, 0)),
            pl.BlockSpec((B, NCLS), lambda: (0, 0)),
        ],
        out_shape=[
            jax.ShapeDtypeStruct((1, 1), jnp.float32),
            jax.ShapeDtypeStruct((B, NCLS), jnp.float32),
        ],
    )(sel, stats, gam2, bet2, w0t, w1t, fcb2, lab2)


def kernel(features, sentence_mask, init_rel, loop_rel, w_in, w_out, w_loop,
           w_rel, bias, bn_gamma, bn_beta, fc_W, fc_b, index, label,
           edges_type, edges):
    f32 = jnp.float32
    i32 = jnp.int32
    pad3 = ((0, 0), (0, 0), (0, PT_PAD - PER_TILE))
    src3 = edges[:, 0].astype(i32).reshape(NCORES, NTILES, PER_TILE)
    dst3 = edges[:, 1].astype(i32).reshape(NCORES, NTILES, PER_TILE)
    typ3 = edges_type.astype(i32).reshape(NCORES, NTILES, PER_TILE)
    src4 = jnp.pad(src3, pad3, constant_values=DUMMY).reshape(
        NCORES, NTILES, ROWS, CH)
    dst4 = jnp.pad(dst3, pad3, constant_values=DUMMY).reshape(
        NCORES, NTILES, ROWS, CH)
    typ4 = jnp.pad(typ3, pad3, constant_values=0).reshape(
        NCORES, NTILES, ROWS, CH)
    xpad = jnp.pad(features.astype(f32), ((0, NPAD - N), (0, 0)))

    deg = _degrees(src4, jnp.ones((CH, D), f32), jnp.zeros((CH, D), f32))
    ytab, dinv = _build(xpad, deg, init_rel.astype(f32).reshape(R, 1, D))
    yflat = ytab.reshape(NCORES * R * NPAD, D)
    accs = _aggregate(yflat, src4, typ4, dst4, jnp.zeros((CH, D), f32))
    outpre, stats = _dense(
        accs[0], accs[1], xpad, dinv[0], dinv[1],
        w_in.astype(f32), w_out.astype(f32), w_loop.astype(f32),
        loop_rel.astype(f32).reshape(1, D), bias.astype(f32).reshape(1, H))
    idx2 = jnp.concatenate([index[0], index[1]]).astype(i32).reshape(
        NCORES * NTILES, SELW)
    sel = _select_rows(outpre, idx2)
    w0t = fc_W[:, :H].T.astype(f32)
    w1t = fc_W[:, H:].T.astype(f32)
    loss2, logits = _head(
        sel, stats, bn_gamma.astype(f32).reshape(1, H),
        bn_beta.astype(f32).reshape(1, H), w0t, w1t,
        fc_b.astype(f32).reshape(1, NCLS), label.astype(i32).reshape(B, 1))
    return loss2[0, 0], logits


# speedup vs baseline: 13.2804x; 1.2007x over previous
import functools

import jax
import jax.numpy as jnp
from jax import lax
from jax.experimental import pallas as pl
from jax.experimental.pallas import tpu as pltpu
from jax.experimental.pallas import tpu_sc as plsc

N = 10000
E = 320000
D = 128
H = 128
NCLS = 16
R = 16
B = 1024

NPAD = 10240
EH = E // 2
NCORES = 2
NTILES = 16
PER_TILE = EH // NTILES
CH = 128
PHASES = 4
PR = 20
PT_PAD = PHASES * PR * CH
OUT_SLICE = NPAD // NTILES
DUMMY = N
RB = 1024
NRB = NPAD // RB
SELW = 2 * B // (NCORES * NTILES)

_mesh = plsc.VectorSubcoreMesh(
    core_axis_name="c", subcore_axis_name="s",
    num_cores=NCORES, num_subcores=NTILES)


@functools.partial(
    pl.kernel,
    out_type=jax.ShapeDtypeStruct((NCORES, NPAD, D), jnp.float32),
    mesh=_mesh,
    scratch_types=[
        pltpu.VMEM_SHARED((NPAD, D), jnp.float32),
        pltpu.VMEM((PR, CH), jnp.int32),
        pltpu.VMEM((CH, D), jnp.float32),
        pltpu.VMEM((CH, D), jnp.float32),
        pltpu.SemaphoreType.DMA,
    ],
)
def _degrees(src_hbm, ones_hbm, zer_hbm, deg_hbm, deg_sp, src_t, ones_v, buf_v,
             sem):
    c = lax.axis_index("c")
    s = lax.axis_index("s")
    pltpu.sync_copy(ones_hbm, ones_v)
    pltpu.sync_copy(zer_hbm, buf_v)
    for q in range(OUT_SLICE // CH):
        pltpu.sync_copy(buf_v, deg_sp.at[pl.ds(s * OUT_SLICE + q * CH, CH)])
    plsc.subcore_barrier()
    for p in range(PHASES):
        pltpu.sync_copy(src_hbm.at[c, s, p], src_t)

        def body(j, carry):
            pltpu.sync_copy(ones_v, deg_sp.at[src_t.at[j]], add=True)
            return carry

        lax.fori_loop(0, PR, body, 0)
    plsc.subcore_barrier()
    for q in range(OUT_SLICE // CH):
        off = s * OUT_SLICE + q * CH
        pltpu.sync_copy(deg_sp.at[pl.ds(off, CH)], buf_v)
        pltpu.sync_copy(buf_v, deg_hbm.at[c, pl.ds(off, CH)])


def _build_body(x_ref, deg_ref, rel_ref, y_ref, dinv_ref):
    d = deg_ref[0][:, 0:1]
    dinv = jnp.where(d > 0.0, lax.rsqrt(d), 0.0)
    dinv_ref[0] = dinv
    y_ref[0, 0] = x_ref[...] * dinv * rel_ref[0]


def _build(xpad, deg, init_rel):
    return pl.pallas_call(
        _build_body,
        grid=(NCORES, R),
        in_specs=[
            pl.BlockSpec((NPAD, D), lambda c, r: (0, 0)),
            pl.BlockSpec((1, NPAD, D), lambda c, r: (c, 0, 0)),
            pl.BlockSpec((1, 1, D), lambda c, r: (r, 0, 0)),
        ],
        out_specs=[
            pl.BlockSpec((1, 1, NPAD, D), lambda c, r: (c, r, 0, 0)),
            pl.BlockSpec((1, NPAD, 1), lambda c, r: (c, 0, 0)),
        ],
        out_shape=[
            jax.ShapeDtypeStruct((NCORES, R, NPAD, D), jnp.float32),
            jax.ShapeDtypeStruct((NCORES, NPAD, 1), jnp.float32),
        ],
    )(xpad, deg, init_rel)


@functools.partial(
    pl.kernel,
    out_type=jax.ShapeDtypeStruct((NCORES, NPAD, D), jnp.float32),
    mesh=_mesh,
    scratch_types=[
        pltpu.VMEM_SHARED((NPAD, D), jnp.float32),
        pltpu.VMEM((PR, CH), jnp.int32),
        pltpu.VMEM((PR, CH), jnp.int32),
        pltpu.VMEM((PR, CH), jnp.int32),
        pltpu.VMEM((CH, D), jnp.float32),
        pltpu.VMEM((CH, D), jnp.float32),
        pltpu.SemaphoreType.DMA,
        pltpu.SemaphoreType.DMA,
    ],
)
def _aggregate(y_hbm, src_hbm, typ_hbm, dst_hbm, zer_hbm, acc_hbm,
               acc_sp, src_t, typ_t, dst_t, rows0, rows1, sem0, sem1):
    c = lax.axis_index("c")
    s = lax.axis_index("s")
    pltpu.sync_copy(zer_hbm, rows0)
    for q in range(OUT_SLICE // CH):
        pltpu.sync_copy(rows0, acc_sp.at[pl.ds(s * OUT_SLICE + q * CH, CH)])
    base = (c * R) * NPAD
    plsc.subcore_barrier()

    for p in range(PHASES):
        pltpu.sync_copy(src_hbm.at[c, s, p], src_t)
        pltpu.sync_copy(typ_hbm.at[c, s, p], typ_t)
        pltpu.sync_copy(dst_hbm.at[c, s, p], dst_t)

        def fidx(j, carry):
            for k in range(CH // 16):
                sl = pl.ds(k * 16, 16)
                src_t[j, sl] = typ_t[j, sl] * NPAD + src_t[j, sl] + base
            return carry

        lax.fori_loop(0, PR, fidx, 0)

        pltpu.async_copy(y_hbm.at[src_t.at[0]], rows0, sem0)

        def chunk2(i, carry):
            j0 = 2 * i
            pltpu.async_copy(y_hbm.at[src_t.at[j0 + 1]], rows1, sem1)
            pltpu.make_async_copy(y_hbm.at[src_t.at[j0]], rows0, sem0).wait()
            pltpu.sync_copy(rows0, acc_sp.at[dst_t.at[j0]], add=True)

            @pl.when(i < PR // 2 - 1)
            def _():
                pltpu.async_copy(y_hbm.at[src_t.at[j0 + 2]], rows0, sem0)

            pltpu.make_async_copy(y_hbm.at[src_t.at[j0 + 1]], rows1, sem1).wait()
            pltpu.sync_copy(rows1, acc_sp.at[dst_t.at[j0 + 1]], add=True)
            return carry

        lax.fori_loop(0, PR // 2, chunk2, 0)
    plsc.subcore_barrier()
    for q in range(OUT_SLICE // CH):
        off = s * OUT_SLICE + q * CH
        pltpu.sync_copy(acc_sp.at[pl.ds(off, CH)], rows0)
        pltpu.sync_copy(rows0, acc_hbm.at[c, pl.ds(off, CH)])


def _dense_body(ain_ref, aout_ref, x_ref, din_ref, dout_ref,
                win_ref, wout_ref, wloop_ref, lrel_ref, bias_ref,
                outpre_ref, stats_ref, sacc):
    i = pl.program_id(0)
    a0 = ain_ref[...] * din_ref[...]
    a1 = aout_ref[...] * dout_ref[...]
    xl = x_ref[...] * lrel_ref[...]
    pre = (jnp.dot(a0, win_ref[...], preferred_element_type=jnp.float32)
           + jnp.dot(a1, wout_ref[...], preferred_element_type=jnp.float32)
           + jnp.dot(xl, wloop_ref[...], preferred_element_type=jnp.float32))
    pre = pre * (1.0 / 3.0) + bias_ref[...]
    outpre_ref[...] = pre
    rowid = i * RB + lax.broadcasted_iota(jnp.int32, (RB, 1), 0)
    prem = jnp.where(rowid < N, pre, 0.0)

    @pl.when(i == 0)
    def _():
        sacc[...] = jnp.zeros((8, 128), jnp.float32)

    s0 = jnp.sum(prem, axis=0, keepdims=True)
    s1 = jnp.sum(prem * prem, axis=0, keepdims=True)
    sacc[0:1, :] = sacc[0:1, :] + s0
    sacc[1:2, :] = sacc[1:2, :] + s1

    @pl.when(i == NRB - 1)
    def _():
        stats_ref[...] = sacc[...]


def _dense(acc_in, acc_out, xpad, dinv_in, dinv_out,
           w_in, w_out, w_loop, loop_rel, bias2):
    return pl.pallas_call(
        _dense_body,
        grid=(NRB,),
        in_specs=[
            pl.BlockSpec((RB, D), lambda i: (i, 0)),
            pl.BlockSpec((RB, D), lambda i: (i, 0)),
            pl.BlockSpec((RB, D), lambda i: (i, 0)),
            pl.BlockSpec((RB, 1), lambda i: (i, 0)),
            pl.BlockSpec((RB, 1), lambda i: (i, 0)),
            pl.BlockSpec((D, H), lambda i: (0, 0)),
            pl.BlockSpec((D, H), lambda i: (0, 0)),
            pl.BlockSpec((D, H), lambda i: (0, 0)),
            pl.BlockSpec((1, D), lambda i: (0, 0)),
            pl.BlockSpec((1, H), lambda i: (0, 0)),
        ],
        out_specs=[
            pl.BlockSpec((RB, H), lambda i: (i, 0)),
            pl.BlockSpec((8, 128), lambda i: (0, 0)),
        ],
        out_shape=[
            jax.ShapeDtypeStruct((NPAD, H), jnp.float32),
            jax.ShapeDtypeStruct((8, 128), jnp.float32),
        ],
        scratch_shapes=[pltpu.VMEM((8, 128), jnp.float32)],
    )(acc_in, acc_out, xpad, dinv_in, dinv_out, w_in, w_out, w_loop,
      loop_rel, bias2)


@functools.partial(
    pl.kernel,
    out_type=jax.ShapeDtypeStruct((2 * B, D), jnp.float32),
    mesh=_mesh,
    scratch_types=[
        pltpu.VMEM((SELW,), jnp.int32),
        pltpu.VMEM((SELW, D), jnp.float32),
        pltpu.SemaphoreType.DMA,
    ],
)
def _select_rows(xpre_hbm, idx_hbm, sel_hbm, idx_v, rows_v, sem):
    c = lax.axis_index("c")
    s = lax.axis_index("s")
    w = s * NCORES + c
    pltpu.sync_copy(idx_hbm.at[w], idx_v)
    pltpu.async_copy(xpre_hbm.at[idx_v], rows_v, sem).wait()
    pltpu.sync_copy(rows_v, sel_hbm.at[pl.ds(w * SELW, SELW)])


def _head_body(sel_ref, stats_ref, gam_ref, bet_ref, w0_ref, w1_ref,
               fcb_ref, lab_ref, loss_ref, logits_ref):
    mean = stats_ref[0:1, :] * (1.0 / N)
    var = stats_ref[1:2, :] * (1.0 / N) - mean * mean
    rs = lax.rsqrt(var + 1e-5)
    xt = jnp.tanh((sel_ref[...] - mean) * rs * gam_ref[...] + bet_ref[...])
    h0 = xt[0:B]
    h1 = xt[B:2 * B]
    lg = (jnp.dot(h0, w0_ref[...], preferred_element_type=jnp.float32)
          + jnp.dot(h1, w1_ref[...], preferred_element_type=jnp.float32)
          + fcb_ref[...])
    m = jnp.max(lg, axis=1, keepdims=True)
    lse = m + jnp.log(jnp.sum(jnp.exp(lg - m), axis=1, keepdims=True))
    lp = lg - lse
    onehot = lax.broadcasted_iota(jnp.int32, (B, NCLS), 1) == lab_ref[...]
    loss_ref[...] = (-jnp.sum(jnp.where(onehot, lp, 0.0)) * (1.0 / B)
                     ).reshape(1, 1)
    logits_ref[...] = lg


def _head(sel, stats, gam2, bet2, w0t, w1t, fcb2, lab2):
    return pl.pallas_call(
        _head_body,
        in_specs=[
            pl.BlockSpec((2 * B, D), lambda: (0, 0)),
            pl.BlockSpec((8, 128), lambda: (0, 0)),
            pl.BlockSpec((1, H), lambda: (0, 0)),
            pl.BlockSpec((1, H), lambda: (0, 0)),
            pl.BlockSpec((H, NCLS), lambda: (0, 0)),
            pl.BlockSpec((H, NCLS), lambda: (0, 0)),
            pl.BlockSpec((1, NCLS), lambda: (0, 0)),
            pl.BlockSpec((B, 1), lambda: (0, 0)),
        ],
        out_specs=[
            pl.BlockSpec((1, 1), lambda: (0, 0)),
            pl.BlockSpec((B, NCLS), lambda: (0, 0)),
        ],
        out_shape=[
            jax.ShapeDtypeStruct((1, 1), jnp.float32),
            jax.ShapeDtypeStruct((B, NCLS), jnp.float32),
        ],
    )(sel, stats, gam2, bet2, w0t, w1t, fcb2, lab2)


def kernel(features, sentence_mask, init_rel, loop_rel, w_in, w_out, w_loop,
           w_rel, bias, bn_gamma, bn_beta, fc_W, fc_b, index, label,
           edges_type, edges):
    f32 = jnp.float32
    i32 = jnp.int32
    pad3 = ((0, 0), (0, 0), (0, PT_PAD - PER_TILE))
    src3 = edges[:, 0].astype(i32).reshape(NCORES, NTILES, PER_TILE)
    dst3 = edges[:, 1].astype(i32).reshape(NCORES, NTILES, PER_TILE)
    typ3 = edges_type.astype(i32).reshape(NCORES, NTILES, PER_TILE)
    src4 = jnp.pad(src3, pad3, constant_values=DUMMY).reshape(
        NCORES, NTILES, PHASES, PR, CH)
    dst4 = jnp.pad(dst3, pad3, constant_values=DUMMY).reshape(
        NCORES, NTILES, PHASES, PR, CH)
    typ4 = jnp.pad(typ3, pad3, constant_values=0).reshape(
        NCORES, NTILES, PHASES, PR, CH)
    xpad = jnp.pad(features.astype(f32), ((0, NPAD - N), (0, 0)))

    deg = _degrees(src4, jnp.ones((CH, D), f32), jnp.zeros((CH, D), f32))
    ytab, dinv = _build(xpad, deg, init_rel.astype(f32).reshape(R, 1, D))
    yflat = ytab.reshape(NCORES * R * NPAD, D)
    accs = _aggregate(yflat, src4, typ4, dst4, jnp.zeros((CH, D), f32))
    outpre, stats = _dense(
        accs[0], accs[1], xpad, dinv[0], dinv[1],
        w_in.astype(f32), w_out.astype(f32), w_loop.astype(f32),
        loop_rel.astype(f32).reshape(1, D), bias.astype(f32).reshape(1, H))
    idx2 = jnp.concatenate([index[0], index[1]]).astype(i32).reshape(
        NCORES * NTILES, SELW)
    sel = _select_rows(outpre, idx2)
    w0t = fc_W[:, :H].T.astype(f32)
    w1t = fc_W[:, H:].T.astype(f32)
    loss2, logits = _head(
        sel, stats, bn_gamma.astype(f32).reshape(1, H),
        bn_beta.astype(f32).reshape(1, H), w0t, w1t,
        fc_b.astype(f32).reshape(1, NCLS), label.astype(i32).reshape(B, 1))
    return loss2[0, 0], logits


# speedup vs baseline: 14.1027x; 1.0619x over previous
import functools

import jax
import jax.numpy as jnp
from jax import lax
from jax.experimental import pallas as pl
from jax.experimental.pallas import tpu as pltpu
from jax.experimental.pallas import tpu_sc as plsc

N = 10000
E = 320000
D = 128
H = 128
NCLS = 16
R = 16
B = 1024

NPAD = 10240
EH = E // 2
NCORES = 2
NTILES = 16
PER_TILE = EH // NTILES
CH = 128
ROWS = (PER_TILE + CH - 1) // CH
PT_PAD = ROWS * CH
OUT_SLICE = NPAD // NTILES
DUMMY = N
RB = 1024
NRB = NPAD // RB
SELW = 2 * B // (NCORES * NTILES)

_mesh = plsc.VectorSubcoreMesh(
    core_axis_name="c", subcore_axis_name="s",
    num_cores=NCORES, num_subcores=NTILES)


@functools.partial(
    pl.kernel,
    out_type=jax.ShapeDtypeStruct((NCORES, NPAD, D), jnp.float32),
    mesh=_mesh,
    scratch_types=[
        pltpu.VMEM_SHARED((NPAD, D), jnp.float32),
        pltpu.VMEM((ROWS, CH), jnp.int32),
        pltpu.VMEM((CH, D), jnp.float32),
        pltpu.VMEM((CH, D), jnp.float32),
    ],
)
def _degrees(src_hbm, ones_hbm, zer_hbm, deg_hbm, deg_sp, src_t, ones_v, buf_v):
    c = lax.axis_index("c")
    s = lax.axis_index("s")
    pltpu.sync_copy(src_hbm.at[c, s], src_t)
    pltpu.sync_copy(ones_hbm, ones_v)
    pltpu.sync_copy(zer_hbm, buf_v)
    for q in range(OUT_SLICE // CH):
        pltpu.sync_copy(buf_v, deg_sp.at[pl.ds(s * OUT_SLICE + q * CH, CH)])
    plsc.subcore_barrier()

    def body(j, carry):
        pltpu.sync_copy(ones_v, deg_sp.at[src_t.at[j]], add=True)
        return carry

    lax.fori_loop(0, ROWS, body, 0)
    plsc.subcore_barrier()
    for q in range(OUT_SLICE // CH):
        off = s * OUT_SLICE + q * CH
        pltpu.sync_copy(deg_sp.at[pl.ds(off, CH)], buf_v)
        pltpu.sync_copy(buf_v, deg_hbm.at[c, pl.ds(off, CH)])


def _build_body(x_ref, deg_ref, rel_ref, y_ref, dinv_ref):
    d = deg_ref[0][:, 0:1]
    dinv = jnp.where(d > 0.0, lax.rsqrt(d), 0.0)
    dinv_ref[0] = dinv
    y_ref[0, 0] = x_ref[...] * dinv * rel_ref[0]


def _build(xpad, deg, init_rel):
    return pl.pallas_call(
        _build_body,
        grid=(NCORES, R),
        in_specs=[
            pl.BlockSpec((NPAD, D), lambda c, r: (0, 0)),
            pl.BlockSpec((1, NPAD, D), lambda c, r: (c, 0, 0)),
            pl.BlockSpec((1, 1, D), lambda c, r: (r, 0, 0)),
        ],
        out_specs=[
            pl.BlockSpec((1, 1, NPAD, D), lambda c, r: (c, r, 0, 0)),
            pl.BlockSpec((1, NPAD, 1), lambda c, r: (c, 0, 0)),
        ],
        out_shape=[
            jax.ShapeDtypeStruct((NCORES, R, NPAD, D), jnp.float32),
            jax.ShapeDtypeStruct((NCORES, NPAD, 1), jnp.float32),
        ],
    )(xpad, deg, init_rel)


@functools.partial(
    pl.kernel,
    out_type=jax.ShapeDtypeStruct((NCORES, NPAD, D), jnp.float32),
    mesh=_mesh,
    scratch_types=[
        pltpu.VMEM_SHARED((NPAD, D), jnp.float32),
        pltpu.VMEM((ROWS, CH), jnp.int32),
        pltpu.VMEM((ROWS, CH), jnp.int32),
        pltpu.VMEM((ROWS, CH), jnp.int32),
        pltpu.VMEM((CH, D), jnp.float32),
        pltpu.SemaphoreType.DMA,
    ],
)
def _aggregate(y_hbm, src_hbm, typ_hbm, dst_hbm, zer_hbm, acc_hbm,
               acc_sp, src_t, typ_t, dst_t, rows0, sem0):
    c = lax.axis_index("c")
    s = lax.axis_index("s")
    pltpu.sync_copy(src_hbm.at[c, s], src_t)
    pltpu.sync_copy(typ_hbm.at[c, s], typ_t)
    pltpu.sync_copy(dst_hbm.at[c, s], dst_t)
    pltpu.sync_copy(zer_hbm, rows0)
    for q in range(OUT_SLICE // CH):
        pltpu.sync_copy(rows0, acc_sp.at[pl.ds(s * OUT_SLICE + q * CH, CH)])
    base = (c * R) * NPAD

    def fidx(j, carry):
        for k in range(CH // 16):
            sl = pl.ds(k * 16, 16)
            src_t[j, sl] = typ_t[j, sl] * NPAD + src_t[j, sl] + base
        return carry

    lax.fori_loop(0, ROWS, fidx, 0)
    plsc.subcore_barrier()

    def chunk(j, carry):
        pltpu.async_copy(y_hbm.at[src_t.at[j]], rows0, sem0).wait()
        pltpu.sync_copy(rows0, acc_sp.at[dst_t.at[j]], add=True)
        return carry

    lax.fori_loop(0, ROWS, chunk, 0)
    plsc.subcore_barrier()
    for q in range(OUT_SLICE // CH):
        off = s * OUT_SLICE + q * CH
        pltpu.sync_copy(acc_sp.at[pl.ds(off, CH)], rows0)
        pltpu.sync_copy(rows0, acc_hbm.at[c, pl.ds(off, CH)])


def _dense_body(ain_ref, aout_ref, x_ref, din_ref, dout_ref,
                win_ref, wout_ref, wloop_ref, lrel_ref, bias_ref,
                outpre_ref, stats_ref, sacc):
    i = pl.program_id(0)
    a0 = ain_ref[...] * din_ref[...]
    a1 = aout_ref[...] * dout_ref[...]
    xl = x_ref[...] * lrel_ref[...]
    pre = (jnp.dot(a0, win_ref[...], preferred_element_type=jnp.float32)
           + jnp.dot(a1, wout_ref[...], preferred_element_type=jnp.float32)
           + jnp.dot(xl, wloop_ref[...], preferred_element_type=jnp.float32))
    pre = pre * (1.0 / 3.0) + bias_ref[...]
    outpre_ref[...] = pre
    rowid = i * RB + lax.broadcasted_iota(jnp.int32, (RB, 1), 0)
    prem = jnp.where(rowid < N, pre, 0.0)

    @pl.when(i == 0)
    def _():
        sacc[...] = jnp.zeros((8, 128), jnp.float32)

    s0 = jnp.sum(prem, axis=0, keepdims=True)
    s1 = jnp.sum(prem * prem, axis=0, keepdims=True)
    sacc[0:1, :] = sacc[0:1, :] + s0
    sacc[1:2, :] = sacc[1:2, :] + s1

    @pl.when(i == NRB - 1)
    def _():
        stats_ref[...] = sacc[...]


def _dense(acc_in, acc_out, xpad, dinv_in, dinv_out,
           w_in, w_out, w_loop, loop_rel, bias2):
    return pl.pallas_call(
        _dense_body,
        grid=(NRB,),
        in_specs=[
            pl.BlockSpec((RB, D), lambda i: (i, 0)),
            pl.BlockSpec((RB, D), lambda i: (i, 0)),
            pl.BlockSpec((RB, D), lambda i: (i, 0)),
            pl.BlockSpec((RB, 1), lambda i: (i, 0)),
            pl.BlockSpec((RB, 1), lambda i: (i, 0)),
            pl.BlockSpec((D, H), lambda i: (0, 0)),
            pl.BlockSpec((D, H), lambda i: (0, 0)),
            pl.BlockSpec((D, H), lambda i: (0, 0)),
            pl.BlockSpec((1, D), lambda i: (0, 0)),
            pl.BlockSpec((1, H), lambda i: (0, 0)),
        ],
        out_specs=[
            pl.BlockSpec((RB, H), lambda i: (i, 0)),
            pl.BlockSpec((8, 128), lambda i: (0, 0)),
        ],
        out_shape=[
            jax.ShapeDtypeStruct((NPAD, H), jnp.float32),
            jax.ShapeDtypeStruct((8, 128), jnp.float32),
        ],
        scratch_shapes=[pltpu.VMEM((8, 128), jnp.float32)],
    )(acc_in, acc_out, xpad, dinv_in, dinv_out, w_in, w_out, w_loop,
      loop_rel, bias2)


@functools.partial(
    pl.kernel,
    out_type=jax.ShapeDtypeStruct((2 * B, D), jnp.float32),
    mesh=_mesh,
    scratch_types=[
        pltpu.VMEM((SELW,), jnp.int32),
        pltpu.VMEM((SELW, D), jnp.float32),
        pltpu.SemaphoreType.DMA,
    ],
)
def _select_rows(xpre_hbm, idx_hbm, sel_hbm, idx_v, rows_v, sem):
    c = lax.axis_index("c")
    s = lax.axis_index("s")
    w = s * NCORES + c
    pltpu.sync_copy(idx_hbm.at[w], idx_v)
    pltpu.async_copy(xpre_hbm.at[idx_v], rows_v, sem).wait()
    pltpu.sync_copy(rows_v, sel_hbm.at[pl.ds(w * SELW, SELW)])


def _head_body(sel_ref, stats_ref, gam_ref, bet_ref, w0_ref, w1_ref,
               fcb_ref, lab_ref, loss_ref, logits_ref):
    mean = stats_ref[0:1, :] * (1.0 / N)
    var = stats_ref[1:2, :] * (1.0 / N) - mean * mean
    rs = lax.rsqrt(var + 1e-5)
    xt = jnp.tanh((sel_ref[...] - mean) * rs * gam_ref[...] + bet_ref[...])
    h0 = xt[0:B]
    h1 = xt[B:2 * B]
    lg = (jnp.dot(h0, w0_ref[...], preferred_element_type=jnp.float32)
          + jnp.dot(h1, w1_ref[...], preferred_element_type=jnp.float32)
          + fcb_ref[...])
    m = jnp.max(lg, axis=1, keepdims=True)
    lse = m + jnp.log(jnp.sum(jnp.exp(lg - m), axis=1, keepdims=True))
    lp = lg - lse
    onehot = lax.broadcasted_iota(jnp.int32, (B, NCLS), 1) == lab_ref[...]
    loss_ref[...] = (-jnp.sum(jnp.where(onehot, lp, 0.0)) * (1.0 / B)
                     ).reshape(1, 1)
    logits_ref[...] = lg


def _head(sel, stats, gam2, bet2, w0t, w1t, fcb2, lab2):
    return pl.pallas_call(
        _head_body,
        in_specs=[
            pl.BlockSpec((2 * B, D), lambda: (0, 0)),
            pl.BlockSpec((8, 128), lambda: (0, 0)),
            pl.BlockSpec((1, H), lambda: (0, 0)),
            pl.BlockSpec((1, H), lambda: (0, 0)),
            pl.BlockSpec((H, NCLS), lambda: (0, 0)),
            pl.BlockSpec((H, NCLS), lambda: (0, 0)),
            pl.BlockSpec((1, NCLS), lambda: (0, 0)),
            pl.BlockSpec((B, 1), lambda: (0, 0)),
        ],
        out_specs=[
            pl.BlockSpec((1, 1), lambda: (0, 0)),
            pl.BlockSpec((B, NCLS), lambda: (0, 0)),
        ],
        out_shape=[
            jax.ShapeDtypeStruct((1, 1), jnp.float32),
            jax.ShapeDtypeStruct((B, NCLS), jnp.float32),
        ],
    )(sel, stats, gam2, bet2, w0t, w1t, fcb2, lab2)


def kernel(features, sentence_mask, init_rel, loop_rel, w_in, w_out, w_loop,
           w_rel, bias, bn_gamma, bn_beta, fc_W, fc_b, index, label,
           edges_type, edges):
    f32 = jnp.float32
    i32 = jnp.int32
    pad3 = ((0, 0), (0, 0), (0, PT_PAD - PER_TILE))
    src3 = edges[:, 0].astype(i32).reshape(NCORES, NTILES, PER_TILE)
    dst3 = edges[:, 1].astype(i32).reshape(NCORES, NTILES, PER_TILE)
    typ3 = edges_type.astype(i32).reshape(NCORES, NTILES, PER_TILE)
    src4 = jnp.pad(src3, pad3, constant_values=DUMMY).reshape(
        NCORES, NTILES, ROWS, CH)
    dst4 = jnp.pad(dst3, pad3, constant_values=DUMMY).reshape(
        NCORES, NTILES, ROWS, CH)
    typ4 = jnp.pad(typ3, pad3, constant_values=0).reshape(
        NCORES, NTILES, ROWS, CH)
    xpad = jnp.pad(features.astype(f32), ((0, NPAD - N), (0, 0)))

    deg = _degrees(src4, jnp.ones((CH, D), f32), jnp.zeros((CH, D), f32))
    ytab, dinv = _build(xpad, deg, init_rel.astype(f32).reshape(R, 1, D))
    yflat = ytab.reshape(NCORES * R * NPAD, D)
    accs = _aggregate(yflat, src4, typ4, dst4, jnp.zeros((CH, D), f32))
    outpre, stats = _dense(
        accs[0], accs[1], xpad, dinv[0], dinv[1],
        w_in.astype(f32), w_out.astype(f32), w_loop.astype(f32),
        loop_rel.astype(f32).reshape(1, D), bias.astype(f32).reshape(1, H))
    idx2 = jnp.concatenate([index[0], index[1]]).astype(i32).reshape(
        NCORES * NTILES, SELW)
    sel = _select_rows(outpre, idx2)
    w0t = fc_W[:, :H].T.astype(f32)
    w1t = fc_W[:, H:].T.astype(f32)
    loss2, logits = _head(
        sel, stats, bn_gamma.astype(f32).reshape(1, H),
        bn_beta.astype(f32).reshape(1, H), w0t, w1t,
        fc_b.astype(f32).reshape(1, NCLS), label.astype(i32).reshape(B, 1))
    return loss2[0, 0], logits
